# baseline (speedup 1.0000x reference)
import functools

import jax
import jax.numpy as jnp
import numpy as np
from jax import lax
from jax.experimental import pallas as pl
from jax.experimental.pallas import tpu as pltpu
from jax.experimental.pallas import tpu_sc as plsc

FG_DIMS = [11, 6, 6, 6, 6, 2, 2, 11, 8, 8, 8, 2]
GROUPS = [(0, 7, 5), (8, 9, 10), (1, 2, 6), (3, 4, 11)]
GDIMS = [tuple(FG_DIMS[m] for m in g) for g in GROUPS]
GSIZES = [d0 * d1 * d2 for (d0, d1, d2) in GDIMS]
GBASES = [int(b) for b in np.cumsum([0] + GSIZES)[:4]]
T_ROWS = 904
EMB = 64
NC, NS, L = 2, 16, 16
NW = NC * NS
R = 512
NG = 4


def _fg_kernel(n_rows: int):
    n_tc = n_rows // 128
    tr_stride = n_tc * 1024
    n_chunks = n_rows // R
    cpw = n_chunks // NW
    rem = n_chunks % NW
    mesh = plsc.VectorSubcoreMesh(core_axis_name="c", subcore_axis_name="s")

    @functools.partial(
        pl.kernel,
        out_type=jax.ShapeDtypeStruct((n_rows * EMB,), jnp.float32),
        mesh=mesh,
        scratch_types=[
            pltpu.VMEM((T_ROWS * (EMB // 2),), jnp.int32),
            pltpu.VMEM((R * 12,), jnp.int32),
            pltpu.VMEM((NG * R,), jnp.int32),
            pltpu.VMEM((R * EMB,), jnp.float32),
            pltpu.SemaphoreType.DMA,
            pltpu.SemaphoreType.DMA,
        ],
        compiler_params=pltpu.CompilerParams(needs_layout_passes=False),
    )
    def k(x_hbm, t_hbm, out_hbm, tv, xv, iv, ov, semx, semo):
        wid = lax.axis_index("s") * NC + lax.axis_index("c")
        pltpu.sync_copy(t_hbm, tv)
        iota = lax.iota(jnp.int32, L)
        coffs = [iota + c * L for c in range(EMB // L)]
        trw = (R // 128) * 1024
        pats = [
            (2 * c + (iota >> 3)) * trw + (iota & 7) * 128
            for c in range(EMB // L)
        ]
        chunk0 = wid * cpw + jnp.minimum(wid, rem)
        my_chunks = cpw + jnp.where(wid < rem, 1, 0)

        def chunk_body(j, carry):
            chunk = chunk0 + j
            base = chunk * R
            descs = [
                pltpu.async_copy(
                    x_hbm.at[pl.ds(i * n_rows + base, R)],
                    xv.at[pl.ds(i * R, R)],
                    semx,
                )
                for i in range(12)
            ]
            for d in descs:
                d.wait()

            @plsc.parallel_loop(0, R, step=L)
            def idx_body(r0):
                for g, ((a, b, c), (_, db, dc)) in enumerate(zip(GROUPS, GDIMS)):
                    xa = xv[pl.ds(a * R + r0, L)]
                    xb = xv[pl.ds(b * R + r0, L)]
                    xc = xv[pl.ds(c * R + r0, L)]
                    idx = (xa * db + xb) * dc + xc
                    eidx = (idx + GBASES[g]) * (EMB // 2)
                    plsc.store_scatter(iv, [iota + (g * R + r0)], eidx)

            @plsc.parallel_loop(0, R, step=1, unroll=2)
            def row_body(r):
                sp = [
                    plsc.load_gather(iv, [jnp.full((L,), g * R, jnp.int32) + r])
                    for g in range(NG)
                ]
                rpart = jnp.full((L,), 0, jnp.int32) + ((r >> 7) * 1024 + (r & 127))
                for h in range(2):
                    v = [
                        plsc.bitcast(
                            plsc.load_gather(tv, [sp[g] + coffs[h]]), jnp.bfloat16
                        )
                        for g in range(NG)
                    ]
                    acc = (v[0] + v[1]) + (v[2] + v[3])
                    au = plsc.bitcast(acc, jnp.int32)
                    lo = plsc.bitcast(au << 16, jnp.float32)
                    hi = plsc.bitcast(au & jnp.int32(-65536), jnp.float32)
                    plsc.store_scatter(ov, [rpart + pats[h]], lo)
                    plsc.store_scatter(ov, [rpart + pats[h + 2]], hi)

            odescs = [
                pltpu.async_copy(
                    ov.at[pl.ds(tr * trw, trw)],
                    out_hbm.at[pl.ds(tr * tr_stride + chunk * trw, trw)],
                    semo,
                )
                for tr in range(8)
            ]
            for d in odescs:
                d.wait()
            return carry

        lax.fori_loop(0, my_chunks, chunk_body, 0)

    return k


def kernel(x, W0, W1, W2, W3, W4, W5, W6, W7, W8, W9, W10, W11):
    tables = [W0, W1, W2, W3, W4, W5, W6, W7, W8, W9, W10, W11]
    combined = []
    for a, b, c in GROUPS:
        t3 = (
            tables[a][:, None, None, :]
            + tables[b][None, :, None, :]
            + tables[c][None, None, :, :]
        )
        combined.append(t3.reshape(-1, EMB))
    t = jnp.concatenate(combined, axis=0)
    t = jnp.pad(t, ((0, T_ROWS - t.shape[0]), (0, 0)))
    tb = t.astype(jnp.bfloat16)
    lo = jax.lax.bitcast_convert_type(tb[:, : EMB // 2], jnp.uint16).astype(jnp.int32)
    hi = jax.lax.bitcast_convert_type(tb[:, EMB // 2 :], jnp.uint16).astype(jnp.int32)
    t = (lo | (hi << 16)).reshape(-1)
    n = x.shape[0]
    x = x.astype(jnp.int32)
    xcols = jnp.concatenate([x[:, i] for i in range(12)])
    img = _fg_kernel(n)(xcols, t)
    out = img.reshape(8, n // 128, 8, 128).transpose(1, 3, 0, 2).reshape(n, EMB)
    return out


# speedup vs baseline: 2.2678x; 2.2678x over previous
import functools

import jax
import jax.numpy as jnp
import numpy as np
from jax import lax
from jax.experimental import pallas as pl
from jax.experimental.pallas import tpu as pltpu
from jax.experimental.pallas import tpu_sc as plsc

FG_DIMS = [11, 6, 6, 6, 6, 2, 2, 11, 8, 8, 8, 2]
GROUPS = [(0, 7, 5), (8, 9, 10), (1, 2, 6), (3, 4, 11)]
GDIMS = [tuple(FG_DIMS[m] for m in g) for g in GROUPS]
GSIZES = [d0 * d1 * d2 for (d0, d1, d2) in GDIMS]
GBASES = [int(b) for b in np.cumsum([0] + GSIZES)[:4]]
T_ROWS = 904
EMB = 64
NC, NS, L = 2, 16, 16
NW = NC * NS
R = 512
NG = 4
TW = EMB // 2 + 1


def _fg_kernel(n_rows: int):
    n_tc = n_rows // 128
    tr_stride = n_tc * 1024
    n_chunks = n_rows // R
    cpw = n_chunks // NW
    rem = n_chunks % NW
    mesh = plsc.VectorSubcoreMesh(core_axis_name="c", subcore_axis_name="s")

    @functools.partial(
        pl.kernel,
        out_type=jax.ShapeDtypeStruct((n_rows * EMB,), jnp.float32),
        mesh=mesh,
        scratch_types=[
            pltpu.VMEM((T_ROWS * TW,), jnp.int32),
            pltpu.VMEM((R * 12,), jnp.int32),
            pltpu.VMEM((NG * R,), jnp.int32),
            pltpu.VMEM((R * EMB,), jnp.float32),
            pltpu.SemaphoreType.DMA,
            pltpu.SemaphoreType.DMA,
        ],
        compiler_params=pltpu.CompilerParams(needs_layout_passes=False),
    )
    def k(x_hbm, t_hbm, out_hbm, tv, xv, iv, ov, semx, semo):
        wid = lax.axis_index("s") * NC + lax.axis_index("c")
        pltpu.sync_copy(t_hbm, tv)
        iota = lax.iota(jnp.int32, L)
        trw = (R // 128) * 1024
        chunk0 = wid * cpw + jnp.minimum(wid, rem)
        my_chunks = cpw + jnp.where(wid < rem, 1, 0)

        def chunk_body(j, carry):
            chunk = chunk0 + j
            base = chunk * R
            descs = [
                pltpu.async_copy(
                    x_hbm.at[pl.ds(i * n_rows + base, R)],
                    xv.at[pl.ds(i * R, R)],
                    semx,
                )
                for i in range(12)
            ]
            for d in descs:
                d.wait()

            @plsc.parallel_loop(0, R, step=L)
            def idx_body(r0):
                for g, ((a, b, c), (_, db, dc)) in enumerate(zip(GROUPS, GDIMS)):
                    xa = xv[pl.ds(a * R + r0, L)]
                    xb = xv[pl.ds(b * R + r0, L)]
                    xc = xv[pl.ds(c * R + r0, L)]
                    idx = (xa * db + xb) * dc + xc
                    eidx = (idx + GBASES[g]) * TW
                    plsc.store_scatter(iv, [iota + (g * R + r0)], eidx)

            @plsc.parallel_loop(0, R, step=L)
            def row_body(r0):
                sp = [iv[pl.ds(g * R + r0, L)] for g in range(NG)]
                rbase = (r0 >> 7) * 1024 + (r0 & 127)
                for k in range(EMB // 2):
                    v = [
                        plsc.bitcast(
                            plsc.load_gather(tv, [sp[g] + k]), jnp.bfloat16
                        )
                        for g in range(NG)
                    ]
                    acc = (v[0] + v[1]) + (v[2] + v[3])
                    au = plsc.bitcast(acc, jnp.int32)
                    lo = plsc.bitcast(au << 16, jnp.float32)
                    hi = plsc.bitcast(au & jnp.int32(-65536), jnp.float32)
                    d2 = k + EMB // 2
                    ov[pl.ds(rbase + (k >> 3) * trw + (k & 7) * 128, L)] = lo
                    ov[pl.ds(rbase + (d2 >> 3) * trw + (d2 & 7) * 128, L)] = hi

            odescs = [
                pltpu.async_copy(
                    ov.at[pl.ds(tr * trw, trw)],
                    out_hbm.at[pl.ds(tr * tr_stride + chunk * trw, trw)],
                    semo,
                )
                for tr in range(8)
            ]
            for d in odescs:
                d.wait()
            return carry

        lax.fori_loop(0, my_chunks, chunk_body, 0)

    return k


def kernel(x, W0, W1, W2, W3, W4, W5, W6, W7, W8, W9, W10, W11):
    tables = [W0, W1, W2, W3, W4, W5, W6, W7, W8, W9, W10, W11]
    combined = []
    for a, b, c in GROUPS:
        t3 = (
            tables[a][:, None, None, :]
            + tables[b][None, :, None, :]
            + tables[c][None, None, :, :]
        )
        combined.append(t3.reshape(-1, EMB))
    t = jnp.concatenate(combined, axis=0)
    t = jnp.pad(t, ((0, T_ROWS - t.shape[0]), (0, 0)))
    tb = t.astype(jnp.bfloat16)
    lo = jax.lax.bitcast_convert_type(tb[:, : EMB // 2], jnp.uint16).astype(jnp.int32)
    hi = jax.lax.bitcast_convert_type(tb[:, EMB // 2 :], jnp.uint16).astype(jnp.int32)
    t = jnp.pad(lo | (hi << 16), ((0, 0), (0, TW - EMB // 2))).reshape(-1)
    n = x.shape[0]
    x = x.astype(jnp.int32)
    xcols = jnp.concatenate([x[:, i] for i in range(12)])
    img = _fg_kernel(n)(xcols, t)
    out = img.reshape(8, n // 128, 8, 128).transpose(1, 3, 0, 2).reshape(n, EMB)
    return out


# speedup vs baseline: 2.5826x; 1.1388x over previous
import functools

import jax
import jax.numpy as jnp
import numpy as np
from jax import lax
from jax.experimental import pallas as pl
from jax.experimental.pallas import tpu as pltpu
from jax.experimental.pallas import tpu_sc as plsc

FG_DIMS = [11, 6, 6, 6, 6, 2, 2, 11, 8, 8, 8, 2]
GROUPS = [(0, 7, 5), (8, 9, 10), (1, 2, 6), (3, 4, 11)]
GDIMS = [tuple(FG_DIMS[m] for m in g) for g in GROUPS]
GSIZES = [d0 * d1 * d2 for (d0, d1, d2) in GDIMS]
GBASES = [int(b) for b in np.cumsum([0] + GSIZES)[:4]]
T_ROWS = 904
EMB = 64
NC, NS, L = 2, 16, 16
NW = NC * NS
R = 512
NG = 4
TW = EMB // 2 + 1


def _fg_kernel(n_rows: int):
    n_tc = n_rows // 128
    tr_stride = n_tc * 1024
    n_chunks = n_rows // R
    cpw = n_chunks // NW
    rem = n_chunks % NW
    mesh = plsc.VectorSubcoreMesh(core_axis_name="c", subcore_axis_name="s")

    @functools.partial(
        pl.kernel,
        out_type=jax.ShapeDtypeStruct((n_rows * EMB,), jnp.float32),
        mesh=mesh,
        scratch_types=[
            pltpu.VMEM((T_ROWS * TW,), jnp.int32),
            pltpu.VMEM((R * 12,), jnp.int32),
            pltpu.VMEM((NG * R,), jnp.int32),
            pltpu.VMEM((R * EMB,), jnp.float32),
            pltpu.SemaphoreType.DMA,
            pltpu.SemaphoreType.DMA,
        ],
        compiler_params=pltpu.CompilerParams(needs_layout_passes=False),
    )
    def k(x_hbm, t_hbm, out_hbm, tv, xv, iv, ov, semx, semo):
        wid = lax.axis_index("s") * NC + lax.axis_index("c")
        pltpu.sync_copy(t_hbm, tv)
        iota = lax.iota(jnp.int32, L)
        trw = (R // 128) * 1024
        chunk0 = wid * cpw + jnp.minimum(wid, rem)
        my_chunks = cpw + jnp.where(wid < rem, 1, 0)

        def issue_x(chunk):
            base = chunk * R
            for i in range(12):
                pltpu.async_copy(
                    x_hbm.at[pl.ds(i * n_rows + base, R)],
                    xv.at[pl.ds(i * R, R)],
                    semx,
                )

        issue_x(chunk0)

        def chunk_body(j, carry):
            chunk = chunk0 + j
            for i in range(12):
                pltpu.make_async_copy(
                    x_hbm.at[pl.ds(i * R, R)], xv.at[pl.ds(i * R, R)], semx
                ).wait()

            @plsc.parallel_loop(0, R, step=L)
            def idx_body(r0):
                for g, ((a, b, c), (_, db, dc)) in enumerate(zip(GROUPS, GDIMS)):
                    xa = xv[pl.ds(a * R + r0, L)]
                    xb = xv[pl.ds(b * R + r0, L)]
                    xc = xv[pl.ds(c * R + r0, L)]
                    idx = (xa * db + xb) * dc + xc
                    eidx = (idx + GBASES[g]) * TW
                    plsc.store_scatter(iv, [iota + (g * R + r0)], eidx)

            @pl.when(j + 1 < my_chunks)
            def _():
                issue_x(chunk + 1)

            @pl.when(j > 0)
            def _():
                for tr in range(8):
                    pltpu.make_async_copy(
                        ov.at[pl.ds(tr * trw, trw)],
                        out_hbm.at[pl.ds(tr * tr_stride, trw)],
                        semo,
                    ).wait()

            @plsc.parallel_loop(0, R, step=L)
            def row_body(r0):
                sp = [iv[pl.ds(g * R + r0, L)] for g in range(NG)]
                rbase = (r0 >> 7) * 1024 + (r0 & 127)
                for k in range(EMB // 2):
                    v = [
                        plsc.bitcast(
                            plsc.load_gather(tv, [sp[g] + k]), jnp.bfloat16
                        )
                        for g in range(NG)
                    ]
                    acc = (v[0] + v[1]) + (v[2] + v[3])
                    au = plsc.bitcast(acc, jnp.int32)
                    lo = plsc.bitcast(au << 16, jnp.float32)
                    hi = plsc.bitcast(au & jnp.int32(-65536), jnp.float32)
                    d2 = k + EMB // 2
                    ov[pl.ds(rbase + (k >> 3) * trw + (k & 7) * 128, L)] = lo
                    ov[pl.ds(rbase + (d2 >> 3) * trw + (d2 & 7) * 128, L)] = hi

            for tr in range(8):
                pltpu.async_copy(
                    ov.at[pl.ds(tr * trw, trw)],
                    out_hbm.at[pl.ds(tr * tr_stride + chunk * trw, trw)],
                    semo,
                )
            return carry

        lax.fori_loop(0, my_chunks, chunk_body, 0)
        for tr in range(8):
            pltpu.make_async_copy(
                ov.at[pl.ds(tr * trw, trw)],
                out_hbm.at[pl.ds(tr * tr_stride, trw)],
                semo,
            ).wait()

    return k


def kernel(x, W0, W1, W2, W3, W4, W5, W6, W7, W8, W9, W10, W11):
    tables = [W0, W1, W2, W3, W4, W5, W6, W7, W8, W9, W10, W11]
    combined = []
    for a, b, c in GROUPS:
        t3 = (
            tables[a][:, None, None, :]
            + tables[b][None, :, None, :]
            + tables[c][None, None, :, :]
        )
        combined.append(t3.reshape(-1, EMB))
    t = jnp.concatenate(combined, axis=0)
    t = jnp.pad(t, ((0, T_ROWS - t.shape[0]), (0, 0)))
    tb = t.astype(jnp.bfloat16)
    lo = jax.lax.bitcast_convert_type(tb[:, : EMB // 2], jnp.uint16).astype(jnp.int32)
    hi = jax.lax.bitcast_convert_type(tb[:, EMB // 2 :], jnp.uint16).astype(jnp.int32)
    t = jnp.pad(lo | (hi << 16), ((0, 0), (0, TW - EMB // 2))).reshape(-1)
    n = x.shape[0]
    x = x.astype(jnp.int32)
    xcols = jnp.concatenate([x[:, i] for i in range(12)])
    img = _fg_kernel(n)(xcols, t)
    out = img.reshape(8, n // 128, 8, 128).transpose(1, 3, 0, 2).reshape(n, EMB)
    return out
